# baseline (speedup 1.0000x reference)
import functools

import jax
import jax.numpy as jnp
from jax import lax
from jax.experimental import pallas as pl
from jax.experimental.pallas import tpu as pltpu
from jax.experimental.pallas import tpu_sc as plsc

_B = 4096
_L = 200
_NPOI = 100
_NW = 32
_LANES = 16

_GR = 2
_NG = 3 * (_L // _GR)
_AR = 8
_NA = 12 * (_L // _AR)


def _sdep_t_body(mft_ref, mit_ref, w12t_ref, dw12t_ref, ts12t_ref, out_ref):
    mft = mft_ref[...]
    mit = mit_ref[...]
    hi = lax.Precision.HIGHEST
    locs = jnp.dot(w12t_ref[...], mft, preferred_element_type=jnp.float32,
                   precision=hi)
    dwoh = (lax.broadcasted_iota(jnp.int32, (7, _B), 0) == mit[0:1, :]
            ).astype(jnp.float32)
    tsoh = (lax.broadcasted_iota(jnp.int32, (48, _B), 0) == mit[1:2, :]
            ).astype(jnp.float32)
    out_ref[...] = (locs
                    + jnp.dot(dw12t_ref[...], dwoh,
                              preferred_element_type=jnp.float32, precision=hi)
                    + jnp.dot(ts12t_ref[...], tsoh,
                              preferred_element_type=jnp.float32, precision=hi))


def _score_body(s2_ref, s3_ref, s4_ref, out_ref):
    out_ref[0] = 1.0 / (s2_ref[0] + 1.0)
    out_ref[1] = s3_ref[0]
    out_ref[2] = s4_ref[0]


def _seqinit_body(sdep_ref, sf_ref, seq_sc_ref, out_ref):
    j = pl.program_id(1)

    @pl.when(j < 12)
    def _():
        row = sdep_ref[pl.ds(j, 1), :]
        out_ref[0] = jnp.broadcast_to(row, (40, _B))

    @pl.when(j >= 12)
    def _():
        out_ref[0] = sf_ref[0]


def _seqinit_kernel(sdep_t, sf_t, seq_sc):
    blk = (1, 40, _B)
    return pl.pallas_call(
        _seqinit_body,
        grid=(_L // 40, 17),
        in_specs=[
            pl.BlockSpec((12, _B), lambda g, j: (0, 0)),
            pl.BlockSpec(blk, lambda g, j: (jnp.maximum(j - 12, 0), g, 0)),
            pl.BlockSpec(memory_space=pl.ANY),
        ],
        out_specs=pl.BlockSpec(
            blk, lambda g, j: (jnp.where(j < 12, j, j + 3), g, 0)),
        out_shape=jax.ShapeDtypeStruct((20, _L, _B), jnp.float32),
        input_output_aliases={2: 0},
    )(sdep_t, sf_t, seq_sc)


_SCORE_ROWS = 40


def _score_kernel(sf_t):
    grid = _L // _SCORE_ROWS
    blk = (1, _SCORE_ROWS, _B)
    return pl.pallas_call(
        _score_body,
        grid=(grid,),
        in_specs=[
            pl.BlockSpec(blk, lambda g: (2, g, 0)),
            pl.BlockSpec(blk, lambda g: (3, g, 0)),
            pl.BlockSpec(blk, lambda g: (4, g, 0)),
        ],
        out_specs=pl.BlockSpec((3, _SCORE_ROWS, _B), lambda g: (0, g, 0)),
        out_shape=jax.ShapeDtypeStruct((3, _L, _B), jnp.float32),
    )(sf_t, sf_t, sf_t)


def _sc_build():
    mesh = plsc.VectorSubcoreMesh(core_axis_name="c", subcore_axis_name="s")
    scratch = (
        [pltpu.VMEM((3, _NPOI + 1), jnp.float32)]
        + [pltpu.VMEM((_GR * _B,), jnp.int32) for _ in range(2)]
        + [pltpu.VMEM((_GR, _B), jnp.float32) for _ in range(2)]
        + [pltpu.SemaphoreType.DMA for _ in range(4)]
    )

    @functools.partial(
        pl.kernel, mesh=mesh,
        out_type=jax.ShapeDtypeStruct((20, _L, _B), jnp.float32),
        scratch_types=scratch,
        compiler_params=pltpu.CompilerParams(needs_layout_passes=False))
    def sc(si_hbm, lpt_hbm, seqT_hbm,
           lpt_v, sibuf0, sibuf1, gout0, gout1,
           gsem0, gsem1, gosem0, gosem1):
        sibufs = (sibuf0, sibuf1)
        gouts = (gout0, gout1)
        gsems = (gsem0, gsem1)
        gosems = (gosem0, gosem1)
        cid = lax.axis_index("c")
        sid = lax.axis_index("s")
        wid = sid * 2 + cid

        pltpu.sync_copy(lpt_hbm, lpt_v)

        n_g = (_NG - wid + _NW - 1) // _NW

        def gpar(q):
            return q // (_L // _GR), 12 + q // (_L // _GR), (q % (_L // _GR)) * _GR

        def g_in(q, p):
            _, _, l0 = gpar(q)
            pltpu.async_copy(si_hbm.at[pl.ds(l0 * _B, _GR * _B)], sibufs[p],
                             gsems[p])

        def g_wait_out(p):
            pltpu.make_async_copy(gouts[p], seqT_hbm.at[0, pl.ds(0, _GR), :],
                                  gosems[p]).wait()

        for p in range(2):
            @pl.when(p < n_g)
            def _(p=p):
                g_in(wid + _NW * p, p)

        def g_step(i, p):
            q = wid + _NW * i
            lp_row, dstp, l0 = gpar(q)
            pltpu.make_async_copy(si_hbm.at[pl.ds(0, _GR * _B)], sibufs[p],
                                  gsems[p]).wait()

            @pl.when(i >= 2)
            def _():
                g_wait_out(p)

            rowv = jnp.full((_LANES,), lp_row, jnp.int32)

            def g_body(k, carry):
                for r in range(_GR):
                    for u in range(4):
                        c = (k * 4 + u) * _LANES
                        idx = sibufs[p][pl.ds(r * _B + c, _LANES)] + 1
                        gouts[p][r, pl.ds(c, _LANES)] = plsc.load_gather(
                            lpt_v, [rowv, idx])
                return carry

            lax.fori_loop(0, _B // (4 * _LANES), g_body, 0)
            pltpu.async_copy(gouts[p], seqT_hbm.at[dstp, pl.ds(l0, _GR), :],
                             gosems[p])

            @pl.when(i + 2 < n_g)
            def _():
                g_in(q + 2 * _NW, p)

        def g_outer(t, carry):
            for p in range(2):
                i = 2 * t + p

                @pl.when(i < n_g)
                def _(i=i, p=p):
                    g_step(i, p)
            return carry

        lax.fori_loop(0, (n_g + 1) // 2, g_outer, 0)

        for p in range(2):
            @pl.when(n_g >= p + 1)
            def _(p=p):
                g_wait_out(p)

    return sc


_sc_kernel = _sc_build()


def kernel(meta_int, meta_float, seq_int, seq_float, W_locS, dw_table,
           ts_table, lp_table):
    mi_t = jnp.transpose(meta_int.astype(jnp.int32))
    mf_t = jnp.transpose(jnp.squeeze(meta_float, -1))
    si_f = jnp.reshape(seq_int.astype(jnp.int32), (_L * _B,))
    sf_t = jnp.transpose(seq_float, (2, 0, 1))
    lp_t = jnp.transpose(lp_table)
    w12t = jnp.zeros((12, _NPOI), jnp.float32).at[0:3].set(W_locS)
    dw12t = jnp.zeros((12, 7), jnp.float32).at[3:6].set(dw_table.T)
    ts12t = jnp.zeros((12, 48), jnp.float32).at[6:12].set(ts_table.T)
    sdep_t = pl.pallas_call(
        _sdep_t_body,
        out_shape=jax.ShapeDtypeStruct((12, _B), jnp.float32),
    )(mf_t, mi_t, w12t, dw12t, ts12t)
    score_t = _score_kernel(sf_t)
    seq_sc = _sc_kernel(si_f, lp_t)
    seq_t = _seqinit_kernel(sdep_t, sf_t, seq_sc)
    return (jnp.transpose(sdep_t),
            jnp.transpose(seq_t, (1, 2, 0)),
            jnp.transpose(score_t, (1, 2, 0)))


# speedup vs baseline: 1.5100x; 1.5100x over previous
import functools

import jax
import jax.numpy as jnp
from jax import lax
from jax.experimental import pallas as pl
from jax.experimental.pallas import tpu as pltpu
from jax.experimental.pallas import tpu_sc as plsc

_B = 4096
_L = 200
_NPOI = 100
_NW = 32
_LANES = 16

_GR = 2
_NG = 3 * (_L // _GR)
_AR = 8
_NA = 12 * (_L // _AR)


def _sdep_t_body(mft_ref, mit_ref, w12t_ref, dw12t_ref, ts12t_ref, out_ref):
    mft = mft_ref[...]
    mit = mit_ref[...]
    hi = lax.Precision.HIGHEST
    locs = jnp.dot(w12t_ref[...], mft, preferred_element_type=jnp.float32,
                   precision=hi)
    dwoh = (lax.broadcasted_iota(jnp.int32, (7, _B), 0) == mit[0:1, :]
            ).astype(jnp.float32)
    tsoh = (lax.broadcasted_iota(jnp.int32, (48, _B), 0) == mit[1:2, :]
            ).astype(jnp.float32)
    out_ref[...] = (locs
                    + jnp.dot(dw12t_ref[...], dwoh,
                              preferred_element_type=jnp.float32, precision=hi)
                    + jnp.dot(ts12t_ref[...], tsoh,
                              preferred_element_type=jnp.float32, precision=hi))


def _score_body(s2_ref, s3_ref, s4_ref, out_ref):
    out_ref[0] = 1.0 / (s2_ref[0] + 1.0)
    out_ref[1] = s3_ref[0]
    out_ref[2] = s4_ref[0]


def _seqinit_body(sdep_ref, sf_ref, seq_sc_ref, out_ref):
    j = pl.program_id(1)

    @pl.when(j < 12)
    def _():
        row = sdep_ref[pl.ds(j, 1), :]
        out_ref[0] = jnp.broadcast_to(row, (40, _B))

    @pl.when(j >= 12)
    def _():
        out_ref[0] = sf_ref[0]


def _seqinit_kernel(sdep_t, sf_t, seq_sc):
    blk = (1, 40, _B)
    return pl.pallas_call(
        _seqinit_body,
        grid=(_L // 40, 17),
        in_specs=[
            pl.BlockSpec((12, _B), lambda g, j: (0, 0)),
            pl.BlockSpec(blk, lambda g, j: (jnp.maximum(j - 12, 0), g, 0)),
            pl.BlockSpec(memory_space=pl.ANY),
        ],
        out_specs=pl.BlockSpec(
            blk, lambda g, j: (jnp.where(j < 12, j, j + 3), g, 0)),
        out_shape=jax.ShapeDtypeStruct((20, _L, _B), jnp.float32),
        input_output_aliases={2: 0},
    )(sdep_t, sf_t, seq_sc)


_SCORE_ROWS = 40


def _score_kernel(sf_t):
    grid = _L // _SCORE_ROWS
    blk = (1, _SCORE_ROWS, _B)
    return pl.pallas_call(
        _score_body,
        grid=(grid,),
        in_specs=[
            pl.BlockSpec(blk, lambda g: (2, g, 0)),
            pl.BlockSpec(blk, lambda g: (3, g, 0)),
            pl.BlockSpec(blk, lambda g: (4, g, 0)),
        ],
        out_specs=pl.BlockSpec((3, _SCORE_ROWS, _B), lambda g: (0, g, 0)),
        out_shape=jax.ShapeDtypeStruct((3, _L, _B), jnp.float32),
    )(sf_t, sf_t, sf_t)


def _sc_build():
    mesh = plsc.VectorSubcoreMesh(core_axis_name="c", subcore_axis_name="s")
    scratch = (
        [pltpu.VMEM((3, _NPOI + 1), jnp.float32)]
        + [pltpu.VMEM((_GR * _B,), jnp.int32) for _ in range(2)]
        + [pltpu.VMEM((_GR, _B), jnp.float32) for _ in range(2)]
        + [pltpu.SemaphoreType.DMA for _ in range(4)]
    )

    @functools.partial(
        pl.kernel, mesh=mesh,
        out_type=jax.ShapeDtypeStruct((20, _L, _B), jnp.float32),
        scratch_types=scratch,
        compiler_params=pltpu.CompilerParams(needs_layout_passes=False))
    def sc(si_hbm, lpt_hbm, seqT_hbm,
           lpt_v, sibuf0, sibuf1, gout0, gout1,
           gsem0, gsem1, gosem0, gosem1):
        sibufs = (sibuf0, sibuf1)
        gouts = (gout0, gout1)
        gsems = (gsem0, gsem1)
        gosems = (gosem0, gosem1)
        cid = lax.axis_index("c")
        sid = lax.axis_index("s")
        wid = sid * 2 + cid

        pltpu.sync_copy(lpt_hbm, lpt_v)

        n_g = (_NG - wid + _NW - 1) // _NW

        def gpar(q):
            return q // (_L // _GR), 12 + q // (_L // _GR), (q % (_L // _GR)) * _GR

        def g_in(q, p):
            _, _, l0 = gpar(q)
            pltpu.async_copy(si_hbm.at[pl.ds(l0 * _B, _GR * _B)], sibufs[p],
                             gsems[p])

        def g_wait_out(p):
            pltpu.make_async_copy(gouts[p], seqT_hbm.at[0, pl.ds(0, _GR), :],
                                  gosems[p]).wait()

        for p in range(2):
            @pl.when(p < n_g)
            def _(p=p):
                g_in(wid + _NW * p, p)

        def g_step(i, p):
            q = wid + _NW * i
            lp_row, dstp, l0 = gpar(q)
            pltpu.make_async_copy(si_hbm.at[pl.ds(0, _GR * _B)], sibufs[p],
                                  gsems[p]).wait()

            @pl.when(i >= 2)
            def _():
                g_wait_out(p)

            rowv = jnp.full((_LANES,), lp_row, jnp.int32)

            for r in range(_GR):
                @plsc.parallel_loop(0, _B, _LANES, unroll=8)
                def _(c, r=r):
                    idx = sibufs[p][pl.ds(r * _B + c, _LANES)] + 1
                    gouts[p][r, pl.ds(c, _LANES)] = plsc.load_gather(
                        lpt_v, [rowv, idx])
            pltpu.async_copy(gouts[p], seqT_hbm.at[dstp, pl.ds(l0, _GR), :],
                             gosems[p])

            @pl.when(i + 2 < n_g)
            def _():
                g_in(q + 2 * _NW, p)

        def g_outer(t, carry):
            for p in range(2):
                i = 2 * t + p

                @pl.when(i < n_g)
                def _(i=i, p=p):
                    g_step(i, p)
            return carry

        lax.fori_loop(0, (n_g + 1) // 2, g_outer, 0)

        for p in range(2):
            @pl.when(n_g >= p + 1)
            def _(p=p):
                g_wait_out(p)

    return sc


_sc_kernel = _sc_build()


def kernel(meta_int, meta_float, seq_int, seq_float, W_locS, dw_table,
           ts_table, lp_table):
    mi_t = jnp.transpose(meta_int.astype(jnp.int32))
    mf_t = jnp.transpose(jnp.squeeze(meta_float, -1))
    si_f = jnp.reshape(seq_int.astype(jnp.int32), (_L * _B,))
    sf_t = jnp.transpose(seq_float, (2, 0, 1))
    lp_t = jnp.transpose(lp_table)
    w12t = jnp.zeros((12, _NPOI), jnp.float32).at[0:3].set(W_locS)
    dw12t = jnp.zeros((12, 7), jnp.float32).at[3:6].set(dw_table.T)
    ts12t = jnp.zeros((12, 48), jnp.float32).at[6:12].set(ts_table.T)
    sdep_t = pl.pallas_call(
        _sdep_t_body,
        out_shape=jax.ShapeDtypeStruct((12, _B), jnp.float32),
    )(mf_t, mi_t, w12t, dw12t, ts12t)
    score_t = _score_kernel(sf_t)
    seq_sc = _sc_kernel(si_f, lp_t)
    seq_t = _seqinit_kernel(sdep_t, sf_t, seq_sc)
    return (jnp.transpose(sdep_t),
            jnp.transpose(seq_t, (1, 2, 0)),
            jnp.transpose(score_t, (1, 2, 0)))


# speedup vs baseline: 1.7026x; 1.1276x over previous
import functools

import jax
import jax.numpy as jnp
from jax import lax
from jax.experimental import pallas as pl
from jax.experimental.pallas import tpu as pltpu
from jax.experimental.pallas import tpu_sc as plsc

_B = 4096
_L = 200
_NPOI = 100
_NW = 32
_LANES = 16

_GR = 2
_NG = 3 * (_L // _GR)
_AR = 8
_NA = 12 * (_L // _AR)


def _sdep_t_body(mft_ref, mit_ref, w12t_ref, dw12t_ref, ts12t_ref, out_ref):
    mft = mft_ref[...]
    mit = mit_ref[...]
    hi = lax.Precision.HIGHEST
    locs = jnp.dot(w12t_ref[...], mft, preferred_element_type=jnp.float32,
                   precision=hi)
    dwoh = (lax.broadcasted_iota(jnp.int32, (7, _B), 0) == mit[0:1, :]
            ).astype(jnp.float32)
    tsoh = (lax.broadcasted_iota(jnp.int32, (48, _B), 0) == mit[1:2, :]
            ).astype(jnp.float32)
    out_ref[...] = (locs
                    + jnp.dot(dw12t_ref[...], dwoh,
                              preferred_element_type=jnp.float32, precision=hi)
                    + jnp.dot(ts12t_ref[...], tsoh,
                              preferred_element_type=jnp.float32, precision=hi))


def _score_body(s2_ref, s3_ref, s4_ref, out_ref):
    out_ref[0] = 1.0 / (s2_ref[0] + 1.0)
    out_ref[1] = s3_ref[0]
    out_ref[2] = s4_ref[0]


def _seqcopy_body(sf_ref, out_ref):
    out_ref[0] = sf_ref[0]


def _seqcopy_kernel(sf_t):
    blk = (1, 40, _B)
    return pl.pallas_call(
        _seqcopy_body,
        grid=(5, _L // 40),
        in_specs=[pl.BlockSpec(blk, lambda j, g: (j, g, 0))],
        out_specs=pl.BlockSpec(blk, lambda j, g: (15 + j, g, 0)),
        out_shape=jax.ShapeDtypeStruct((20, _L, _B), jnp.float32),
    )(sf_t)


_SCORE_ROWS = 40


def _score_kernel(sf_t):
    grid = _L // _SCORE_ROWS
    blk = (1, _SCORE_ROWS, _B)
    return pl.pallas_call(
        _score_body,
        grid=(grid,),
        in_specs=[
            pl.BlockSpec(blk, lambda g: (2, g, 0)),
            pl.BlockSpec(blk, lambda g: (3, g, 0)),
            pl.BlockSpec(blk, lambda g: (4, g, 0)),
        ],
        out_specs=pl.BlockSpec((3, _SCORE_ROWS, _B), lambda g: (0, g, 0)),
        out_shape=jax.ShapeDtypeStruct((3, _L, _B), jnp.float32),
    )(sf_t, sf_t, sf_t)


def _sc_build():
    mesh = plsc.VectorSubcoreMesh(core_axis_name="c", subcore_axis_name="s")
    scratch = (
        [pltpu.VMEM((3, _NPOI + 1), jnp.float32)]
        + [pltpu.VMEM((_GR * _B,), jnp.int32) for _ in range(2)]
        + [pltpu.VMEM((_GR, _B), jnp.float32) for _ in range(2)]
        + [pltpu.VMEM_SHARED((12, _AR, _B), jnp.float32)]
        + [pltpu.SemaphoreType.DMA for _ in range(5)]
    )

    @functools.partial(
        pl.kernel, mesh=mesh,
        out_type=(),
        scratch_types=scratch,
        compiler_params=pltpu.CompilerParams(needs_layout_passes=False))
    def sc(sdepT_hbm, si_hbm, lpt_hbm, seqT_hbm,
           lpt_v, sibuf0, sibuf1, gout0, gout1, sdep_s,
           gsem0, gsem1, gosem0, gosem1, asem):
        sibufs = (sibuf0, sibuf1)
        gouts = (gout0, gout1)
        gsems = (gsem0, gsem1)
        gosems = (gosem0, gosem1)
        cid = lax.axis_index("c")
        sid = lax.axis_index("s")
        wid = sid * 2 + cid

        pltpu.sync_copy(lpt_hbm, lpt_v)

        n_g = (_NG - wid + _NW - 1) // _NW
        n_a = (_NA - wid + _NW - 1) // _NW

        def gpar(q):
            return q // (_L // _GR), 12 + q // (_L // _GR), (q % (_L // _GR)) * _GR

        def g_in(q, p):
            _, _, l0 = gpar(q)
            pltpu.async_copy(si_hbm.at[pl.ds(l0 * _B, _GR * _B)], sibufs[p],
                             gsems[p])

        def g_wait_out(p):
            pltpu.make_async_copy(gouts[p], seqT_hbm.at[0, pl.ds(0, _GR), :],
                                  gosems[p]).wait()

        for p in range(2):
            @pl.when(p < n_g)
            def _(p=p):
                g_in(wid + _NW * p, p)

        @pl.when(sid < 12)
        def _():
            for r in range(_AR):
                pltpu.async_copy(sdepT_hbm.at[sid, :], sdep_s.at[sid, r, :],
                                 asem)
            for r in range(_AR):
                pltpu.make_async_copy(sdepT_hbm.at[0, :],
                                      sdep_s.at[0, 0, :], asem).wait()

        plsc.subcore_barrier()

        def a_body(i, carry):
            q = wid + _NW * i
            j = q // (_L // _AR)
            l0 = (q % (_L // _AR)) * _AR

            @pl.when(i >= 6)
            def _():
                pltpu.make_async_copy(sdep_s.at[0],
                                      seqT_hbm.at[0, pl.ds(0, _AR), :],
                                      asem).wait()

            pltpu.async_copy(sdep_s.at[j], seqT_hbm.at[j, pl.ds(l0, _AR), :],
                             asem)
            return carry

        lax.fori_loop(0, n_a, a_body, 0)

        def g_step(i, p):
            q = wid + _NW * i
            lp_row, dstp, l0 = gpar(q)
            pltpu.make_async_copy(si_hbm.at[pl.ds(0, _GR * _B)], sibufs[p],
                                  gsems[p]).wait()

            @pl.when(i >= 2)
            def _():
                g_wait_out(p)

            rowv = jnp.full((_LANES,), lp_row, jnp.int32)

            for r in range(_GR):
                @plsc.parallel_loop(0, _B, _LANES, unroll=8)
                def _(c, r=r):
                    idx = sibufs[p][pl.ds(r * _B + c, _LANES)] + 1
                    gouts[p][r, pl.ds(c, _LANES)] = plsc.load_gather(
                        lpt_v, [rowv, idx])
            pltpu.async_copy(gouts[p], seqT_hbm.at[dstp, pl.ds(l0, _GR), :],
                             gosems[p])

            @pl.when(i + 2 < n_g)
            def _():
                g_in(q + 2 * _NW, p)

        def g_outer(t, carry):
            for p in range(2):
                i = 2 * t + p

                @pl.when(i < n_g)
                def _(i=i, p=p):
                    g_step(i, p)
            return carry

        lax.fori_loop(0, (n_g + 1) // 2, g_outer, 0)

        for p in range(2):
            @pl.when(n_g >= p + 1)
            def _(p=p):
                g_wait_out(p)

        def a_drain(i, carry):
            pltpu.make_async_copy(sdep_s.at[0],
                                  seqT_hbm.at[0, pl.ds(0, _AR), :],
                                  asem).wait()
            return carry

        lax.fori_loop(0, jnp.minimum(n_a, 6), a_drain, 0)

    return sc


_sc_kernel = _sc_build()


def kernel(meta_int, meta_float, seq_int, seq_float, W_locS, dw_table,
           ts_table, lp_table):
    mi_t = jnp.transpose(meta_int.astype(jnp.int32))
    mf_t = jnp.transpose(jnp.squeeze(meta_float, -1))
    si_f = jnp.reshape(seq_int.astype(jnp.int32), (_L * _B,))
    sf_t = jnp.transpose(seq_float, (2, 0, 1))
    lp_t = jnp.transpose(lp_table)
    w12t = jnp.zeros((12, _NPOI), jnp.float32).at[0:3].set(W_locS)
    dw12t = jnp.zeros((12, 7), jnp.float32).at[3:6].set(dw_table.T)
    ts12t = jnp.zeros((12, 48), jnp.float32).at[6:12].set(ts_table.T)
    sdep_t = pl.pallas_call(
        _sdep_t_body,
        out_shape=jax.ShapeDtypeStruct((12, _B), jnp.float32),
    )(mf_t, mi_t, w12t, dw12t, ts12t)
    score_t = _score_kernel(sf_t)
    seq_init = _seqcopy_kernel(sf_t)
    seq_ref = jax.new_ref(seq_init)
    _sc_kernel(sdep_t, si_f, lp_t, seq_ref)
    seq_t = seq_ref[...]
    return (jnp.transpose(sdep_t),
            jnp.transpose(seq_t, (1, 2, 0)),
            jnp.transpose(score_t, (1, 2, 0)))
